# baseline (speedup 1.0000x reference)
import functools

import jax
import jax.numpy as jnp
from jax import lax
from jax.experimental import pallas as pl
from jax.experimental.pallas import tpu as pltpu
from jax.experimental.pallas import tpu_sc as plsc

N = 10000
E = 320000
D = 128
DE = 16
B = 64
STEPS = 5
NLSTM = 2

NC = 2
NS = 16
NW = NC * NS
RSPAN = 320
K = 80
G = 8
NCHUNK = 160
CAP = NCHUNK * K
ZROWS = 80
LASTROWS = N - (NS * NC - 1) * RSPAN

_GATHER_DNUMS = lax.GatherDimensionNumbers(
    offset_dims=(), collapsed_slice_dims=(0,), start_index_map=(0,))


def _leaky(x):
    return jnp.where(x > 0, x, 0.01 * x)


def _elu(x):
    return jnp.where(x > 0, x, jnp.expm1(x))


def _bn(x, g, b):
    m = jnp.mean(x, axis=0)
    v = jnp.var(x, axis=0)
    return g * (x - m) / jnp.sqrt(v + 1e-5) + b


def _splat(v, j):
    idx = jnp.full((16, 1), j, jnp.int32)
    return lax.gather(v, idx, _GATHER_DNUMS, slice_sizes=(1,),
                      mode=lax.GatherScatterMode.PROMISE_IN_BOUNDS)


def _spmv_body(x_hbm, src_hbm, dst_hbm, pd_hbm, nd_hbm, w_hbm, out_hbm,
               srcb, dstb, pdb, ndb, wb, rows_v, zbuf, acc, gsem):
    cid = lax.axis_index("c")
    sid = lax.axis_index("s")
    wid = cid * NS + sid

    zv = jnp.zeros((16,), jnp.float32)

    def _zero(i, _):
        r = i // (D // 16)
        t = i % (D // 16)
        zbuf[r, pl.ds(t * 16, 16)] = zv
        return _

    lax.fori_loop(0, ZROWS * (D // 16), _zero, None)
    for rep in range(RSPAN // ZROWS):
        pltpu.sync_copy(zbuf, acc.at[pl.ds(sid * RSPAN + rep * ZROWS, ZROWS)])

    nd16 = D // 16

    def _chunk(c, A):
        jg = c // G
        r = c - jg * G

        @pl.when(r == 0)
        def _stage():
            pltpu.sync_copy(src_hbm.at[wid, pl.ds(jg * G, G)], srcb)
            pltpu.sync_copy(dst_hbm.at[wid, pl.ds(jg * G, G)], dstb)
            pltpu.sync_copy(pd_hbm.at[wid, pl.ds(jg * G, G)], pdb)
            pltpu.sync_copy(nd_hbm.at[wid, pl.ds(jg * G, G)], ndb)
            pltpu.sync_copy(w_hbm.at[wid, pl.ds(jg * G, G)], wb)

        pltpu.async_copy(x_hbm.at[srcb.at[r]], rows_v, gsem).wait()

        def _g16(g, A):
            wv = wb[r, pl.ds(g * 16, 16)]
            dv = dstb[r, pl.ds(g * 16, 16)]
            pv = pdb[r, pl.ds(g * 16, 16)]
            nv = ndb[r, pl.ds(g * 16, 16)]
            keepv = jnp.where(dv == pv, 1.0, 0.0)
            lastv = jnp.where(dv == nv, 0.0, 1.0)
            A = list(A)
            for j in range(16):
                sp = _splat(wv, j)
                kf = _splat(keepv, j)
                lf = _splat(lastv, j)
                for d in range(nd16):
                    rv = rows_v[g * 16 + j, pl.ds(d * 16, 16)]
                    a = rv * sp + A[d] * kf
                    A[d] = a
                    rows_v[g * 16 + j, pl.ds(d * 16, 16)] = a * lf
            return tuple(A)

        A = lax.fori_loop(0, K // 16, _g16, A)
        pltpu.sync_copy(rows_v, acc.at[dstb.at[r]], add=True)
        return A

    zero = jnp.zeros((16,), jnp.float32)
    lax.fori_loop(0, NCHUNK, _chunk, tuple(zero for _ in range(nd16)))

    @pl.when(wid < NW - 1)
    def _flush():
        pltpu.sync_copy(acc.at[pl.ds(sid * RSPAN, RSPAN)],
                        out_hbm.at[pl.ds(wid * RSPAN, RSPAN)])

    @pl.when(wid == NW - 1)
    def _flush_last():
        pltpu.sync_copy(acc.at[pl.ds(sid * RSPAN, LASTROWS)],
                        out_hbm.at[pl.ds((NW - 1) * RSPAN, LASTROWS)])


@functools.lru_cache(maxsize=None)
def _get_spmv():
    return pl.kernel(
        _spmv_body,
        out_type=jax.ShapeDtypeStruct((N, D), jnp.float32),
        mesh=plsc.VectorSubcoreMesh(core_axis_name="c", subcore_axis_name="s"),
        scratch_types=[
            pltpu.VMEM((G, K), jnp.int32),
            pltpu.VMEM((G, K), jnp.int32),
            pltpu.VMEM((G, K), jnp.int32),
            pltpu.VMEM((G, K), jnp.int32),
            pltpu.VMEM((G, K), jnp.float32),
            pltpu.VMEM((K, D), jnp.float32),
            pltpu.VMEM((ZROWS, D), jnp.float32),
            pltpu.VMEM_SHARED((NS * RSPAN, D), jnp.float32),
            pltpu.SemaphoreType.DMA,
        ],
    )


def _segment_sum_sc(xin, src_t, dst_t, pd_t, nd_t, w_l):
    return _get_spmv()(xin, src_t, dst_t, pd_t, nd_t, w_l)


def _prep_edges(src, dst, w_all):
    perm = jnp.argsort(dst, stable=True)
    sdst = dst[perm]
    ssrc = src[perm]
    sw = w_all[:, perm]
    owner = sdst // RSPAN
    ptr = jnp.searchsorted(
        sdst, jnp.arange(NW, dtype=jnp.int32) * RSPAN).astype(jnp.int32)
    rank = jnp.arange(E, dtype=jnp.int32) - ptr[owner]
    pos = jnp.where(rank < CAP, owner * CAP + rank, jnp.int32(NW * CAP))
    g = jnp.full((NW * CAP,), E, jnp.int32)
    g = g.at[pos].set(jnp.arange(E, dtype=jnp.int32), mode='drop')
    ssrc_x = jnp.concatenate([ssrc, jnp.zeros((1,), jnp.int32)])
    dsc = sdst - (owner // NS) * (NS * RSPAN)
    dloc_x = jnp.concatenate([dsc, jnp.zeros((1,), jnp.int32)])
    sw_x = jnp.concatenate([sw, jnp.zeros((8, 1), jnp.float32)], axis=1)
    slot = jnp.arange(NW * CAP, dtype=jnp.int32)
    pad_row = ((slot // CAP) % NS) * RSPAN
    pad_src = slot % N
    src_t = jnp.where(g == E, pad_src, ssrc_x[g]).reshape(NW, NCHUNK, K)
    dflat = jnp.where(g == E, pad_row, dloc_x[g]).reshape(NW, CAP)
    w_t = sw_x[:, g].reshape(8, NW, NCHUNK, K)
    m1 = jnp.full((NW, 1), -1, jnp.int32)
    pd = jnp.concatenate([m1, dflat[:, :-1]], axis=1)
    nd = jnp.concatenate([dflat[:, 1:], m1], axis=1)
    dst_t = dflat.reshape(NW, NCHUNK, K)
    pd_t = pd.reshape(NW, NCHUNK, K)
    nd_t = nd.reshape(NW, NCHUNK, K)
    return src_t, dst_t, pd_t, nd_t, w_t


def kernel(x, edge_index, edge_attr, batch, params):
    src = edge_index[0]
    dst = edge_index[1]
    convs = params['convs']

    w_layers = []
    for p in convs:
        h = _leaky(edge_attr @ p['et_W1'] + p['et_b1'])
        w_layers.append(_elu(h @ p['et_W2'] + p['et_b2'])[:, 0])
    w_all = jnp.stack(w_layers)

    src_t, dst_t, pd_t, nd_t, w_t = _prep_edges(src, dst, w_all)

    def conv(xin, l):
        p = convs[l]
        agg = _segment_sum_sc(xin, src_t, dst_t, pd_t, nd_t, w_t[l])
        out = agg + xin
        h1 = _leaky(out @ p['nn_W1'] + p['nn_b1'])
        return h1 @ p['nn_W2'] + p['nn_b2']

    h = conv(x, 0)
    for l in range(7):
        skip = h
        hb = _leaky(_bn(h, params['bns'][l]['gamma'], params['bns'][l]['beta']))
        h = conv(hb, l + 1) + skip
    h = _leaky(h)
    h = _bn(h, params['bn8']['gamma'], params['bn8']['beta'])

    lp = params['lstm']
    q_star = jnp.zeros((B, 2 * D), dtype=h.dtype)
    hs = [jnp.zeros((B, D), dtype=h.dtype) for _ in range(NLSTM)]
    cs = [jnp.zeros((B, D), dtype=h.dtype) for _ in range(NLSTM)]
    for _ in range(STEPS):
        inp = q_star
        for l in range(NLSTM):
            gates = (inp @ lp['W_ih_%d' % l].T + lp['b_ih_%d' % l]
                     + hs[l] @ lp['W_hh_%d' % l].T + lp['b_hh_%d' % l])
            i, f, g, o = jnp.split(gates, 4, axis=-1)
            i = jax.nn.sigmoid(i)
            f = jax.nn.sigmoid(f)
            g = jnp.tanh(g)
            o = jax.nn.sigmoid(o)
            cs[l] = f * cs[l] + i * g
            hs[l] = o * jnp.tanh(cs[l])
            inp = hs[l]
        q = hs[-1]
        e = jnp.sum(h * q[batch], axis=-1)
        emax = jax.ops.segment_max(e, batch, num_segments=B)
        emax = jnp.where(jnp.isfinite(emax), emax, 0.0)
        a = jnp.exp(e - emax[batch])
        denom = jax.ops.segment_sum(a, batch, num_segments=B)
        a = a / (denom[batch] + 1e-16)
        r = jax.ops.segment_sum(a[:, None] * h, batch, num_segments=B)
        q_star = jnp.concatenate([q, r], axis=-1)
    return q_star
